# initial kernel scaffold (unmeasured)
import jax
import jax.numpy as jnp
from jax import lax
from jax.experimental import pallas as pl
from jax.experimental.pallas import tpu as pltpu

N_DEV = 32


def kernel(x, w_mat, scale_x, scale_w):
    m, k = x.shape
    _, n = w_mat.shape
    ch = m // N_DEV

    def body(x_ref, w_ref, sx_ref, sw_ref, out_ref,
             comm_ref, send_sem, recv_sem, credit_sem):
        my = lax.axis_index("i")
        right = jnp.mod(my + 1, N_DEV)
        left = jnp.mod(my - 1, N_DEV)

        wb = w_ref[...].astype(jnp.bfloat16)

        def gemm_chunk(c, _):
            out_ref[pl.ds(c * ch, ch), :] = jnp.dot(
                x_ref[pl.ds(c * ch, ch), :].astype(jnp.bfloat16),
                wb,
                preferred_element_type=jnp.float32,
            )
            return _

        lax.fori_loop(0, N_DEV, gemm_chunk, None)

        bar = pltpu.get_barrier_semaphore()
        for nbr in (left, right):
            pl.semaphore_signal(
                bar, inc=1, device_id=(nbr,),
                device_id_type=pl.DeviceIdType.MESH,
            )
        pl.semaphore_wait(bar, 2)

        def rs_hop(s, _):
            send_idx = jnp.mod(my - s, N_DEV)
            recv_idx = jnp.mod(my - s - 1, N_DEV)

            @pl.when(s > 0)
            def _():
                pl.semaphore_wait(credit_sem, 1)

            rdma = pltpu.make_async_remote_copy(
                src_ref=out_ref.at[pl.ds(send_idx * ch, ch), :],
                dst_ref=comm_ref,
                send_sem=send_sem,
                recv_sem=recv_sem,
                device_id=(right,),
                device_id_type=pl.DeviceIdType.MESH,
            )
            rdma.start()
            rdma.wait()

            out_ref[pl.ds(recv_idx * ch, ch), :] = (
                out_ref[pl.ds(recv_idx * ch, ch), :] + comm_ref[...]
            )

            @pl.when(s < N_DEV - 2)
            def _():
                pl.semaphore_signal(
                    credit_sem, inc=1, device_id=(left,),
                    device_id_type=pl.DeviceIdType.MESH,
                )
            return _

        lax.fori_loop(0, N_DEV - 1, rs_hop, None)

        own = jnp.mod(my + 1, N_DEV)
        scale = sx_ref[0] * sw_ref[0]
        rows = pl.ds(own * ch, ch)
        out_ref[rows, :] = jnp.maximum(out_ref[rows, :] * scale, 0.0)

        def ag_hop(s, _):
            send_idx = jnp.mod(my + 1 - s, N_DEV)

            @pl.when(s > 0)
            def _():
                pl.semaphore_wait(credit_sem, 1)

            rdma = pltpu.make_async_remote_copy(
                src_ref=out_ref.at[pl.ds(send_idx * ch, ch), :],
                dst_ref=out_ref.at[pl.ds(send_idx * ch, ch), :],
                send_sem=send_sem,
                recv_sem=recv_sem,
                device_id=(right,),
                device_id_type=pl.DeviceIdType.MESH,
            )
            rdma.start()
            rdma.wait()

            @pl.when(s < N_DEV - 2)
            def _():
                pl.semaphore_signal(
                    credit_sem, inc=1, device_id=(left,),
                    device_id_type=pl.DeviceIdType.MESH,
                )
            return _

        lax.fori_loop(0, N_DEV - 1, ag_hop, None)

    return pl.pallas_call(
        body,
        out_shape=jax.ShapeDtypeStruct((m, n), jnp.float32),
        in_specs=[
            pl.BlockSpec(memory_space=pltpu.VMEM),
            pl.BlockSpec(memory_space=pltpu.VMEM),
            pl.BlockSpec(memory_space=pltpu.SMEM),
            pl.BlockSpec(memory_space=pltpu.SMEM),
        ],
        out_specs=pl.BlockSpec(memory_space=pltpu.VMEM),
        scratch_shapes=[
            pltpu.VMEM((ch, n), jnp.float32),
            pltpu.SemaphoreType.DMA,
            pltpu.SemaphoreType.DMA,
            pltpu.SemaphoreType.REGULAR,
        ],
        compiler_params=pltpu.CompilerParams(collective_id=0),
    )(x, w_mat, scale_x, scale_w)


# baseline (device time: 1174448 ns/iter reference)
import jax
import jax.numpy as jnp
from jax import lax
from jax.experimental import pallas as pl
from jax.experimental.pallas import tpu as pltpu

N_DEV = 32


def kernel(x, w_mat, scale_x, scale_w):
    m, k = x.shape
    _, n = w_mat.shape
    ch = m // N_DEV

    def body(x_ref, w_ref, sx_ref, sw_ref, out_ref,
             comm_ref, send_sem, recv_sem, credit_sem):
        my = lax.axis_index("i")
        right = jnp.mod(my + 1, N_DEV)
        left = jnp.mod(my - 1, N_DEV)

        wb = w_ref[...].astype(jnp.bfloat16)

        def gemm_chunk(c, _):
            out_ref[pl.ds(c * ch, ch), :] = jnp.dot(
                x_ref[pl.ds(c * ch, ch), :].astype(jnp.bfloat16),
                wb,
                preferred_element_type=jnp.float32,
            )
            return _

        lax.fori_loop(0, N_DEV, gemm_chunk, None)

        bar = pltpu.get_barrier_semaphore()
        for nbr in (left, right):
            pl.semaphore_signal(
                bar, inc=1, device_id=(nbr,),
                device_id_type=pl.DeviceIdType.MESH,
            )
        pl.semaphore_wait(bar, 2)

        def rs_hop(s, _):
            send_idx = jnp.mod(my - s, N_DEV)
            recv_idx = jnp.mod(my - s - 1, N_DEV)

            @pl.when(s > 0)
            def _():
                pl.semaphore_wait(credit_sem, 1)

            rdma = pltpu.make_async_remote_copy(
                src_ref=out_ref.at[pl.ds(send_idx * ch, ch), :],
                dst_ref=comm_ref,
                send_sem=send_sem,
                recv_sem=recv_sem,
                device_id=(right,),
                device_id_type=pl.DeviceIdType.MESH,
            )
            rdma.start()
            rdma.wait()

            out_ref[pl.ds(recv_idx * ch, ch), :] = (
                out_ref[pl.ds(recv_idx * ch, ch), :] + comm_ref[...]
            )

            @pl.when(s < N_DEV - 2)
            def _():
                pl.semaphore_signal(
                    credit_sem, inc=1, device_id=(left,),
                    device_id_type=pl.DeviceIdType.MESH,
                )
            return _

        lax.fori_loop(0, N_DEV - 1, rs_hop, None)

        own = jnp.mod(my + 1, N_DEV)
        scale = sx_ref[0] * sw_ref[0]
        rows = pl.ds(own * ch, ch)
        out_ref[rows, :] = jnp.maximum(out_ref[rows, :] * scale, 0.0)

        def ag_hop(s, _):
            send_idx = jnp.mod(my + 1 - s, N_DEV)

            @pl.when(s > 0)
            def _():
                pl.semaphore_wait(credit_sem, 1)

            rdma = pltpu.make_async_remote_copy(
                src_ref=out_ref.at[pl.ds(send_idx * ch, ch), :],
                dst_ref=out_ref.at[pl.ds(send_idx * ch, ch), :],
                send_sem=send_sem,
                recv_sem=recv_sem,
                device_id=(right,),
                device_id_type=pl.DeviceIdType.MESH,
            )
            rdma.start()
            rdma.wait()

            @pl.when(s < N_DEV - 2)
            def _():
                pl.semaphore_signal(
                    credit_sem, inc=1, device_id=(left,),
                    device_id_type=pl.DeviceIdType.MESH,
                )
            return _

        lax.fori_loop(0, N_DEV - 1, ag_hop, None)

    return pl.pallas_call(
        body,
        out_shape=jax.ShapeDtypeStruct((m, n), jnp.float32),
        in_specs=[
            pl.BlockSpec(memory_space=pltpu.VMEM),
            pl.BlockSpec(memory_space=pltpu.VMEM),
            pl.BlockSpec(memory_space=pltpu.SMEM),
            pl.BlockSpec(memory_space=pltpu.SMEM),
        ],
        out_specs=pl.BlockSpec(memory_space=pltpu.VMEM),
        scratch_shapes=[
            pltpu.VMEM((ch, n), jnp.float32),
            pltpu.SemaphoreType.DMA,
            pltpu.SemaphoreType.DMA,
            pltpu.SemaphoreType.REGULAR,
        ],
        compiler_params=pltpu.CompilerParams(
            collective_id=0,
            vmem_limit_bytes=60 * 1024 * 1024,
        ),
    )(x, w_mat, scale_x, scale_w)


# device time: 665117 ns/iter; 1.7658x vs baseline; 1.7658x over previous
import jax
import jax.numpy as jnp
from jax import lax
from jax.experimental import pallas as pl
from jax.experimental.pallas import tpu as pltpu

N_DEV = 32


def kernel(x, w_mat, scale_x, scale_w):
    m, k = x.shape
    _, n = w_mat.shape
    ch = m // N_DEV
    n2 = n // 2

    def body(x_ref, w_ref, sx_ref, sw_ref, out_ref,
             stage_a, stage_b, rs_comm_a, rs_comm_b, ag_comm_a, ag_comm_b,
             send_sem_a, recv_sem_a, send_sem_b, recv_sem_b,
             credit_a, credit_b):
        my = lax.axis_index("i")
        right = jnp.mod(my + 1, N_DEV)
        left = jnp.mod(my - 1, N_DEV)

        wb = w_ref[...].astype(jnp.bfloat16)

        def gemm_chunk(c, _):
            out_ref[pl.ds(c * ch, ch), :] = jnp.dot(
                x_ref[pl.ds(c * ch, ch), :].astype(jnp.bfloat16),
                wb,
                preferred_element_type=jnp.float32,
            )
            return _

        lax.fori_loop(0, N_DEV, gemm_chunk, None)

        bar = pltpu.get_barrier_semaphore()
        for nbr in (left, right):
            pl.semaphore_signal(
                bar, inc=1, device_id=(nbr,),
                device_id_type=pl.DeviceIdType.MESH,
            )
        pl.semaphore_wait(bar, 2)

        def rs_hop(s, _):
            send_a = jnp.mod(my - s, N_DEV)
            recv_a = jnp.mod(my - s - 1, N_DEV)
            send_b = jnp.mod(my + s, N_DEV)
            recv_b = jnp.mod(my + s + 1, N_DEV)

            stage_a[...] = out_ref[pl.ds(send_a * ch, ch), :n2].astype(
                jnp.bfloat16)
            stage_b[...] = out_ref[pl.ds(send_b * ch, ch), n2:].astype(
                jnp.bfloat16)

            @pl.when(s > 0)
            def _():
                pl.semaphore_wait(credit_a, 1)
                pl.semaphore_wait(credit_b, 1)

            rdma_a = pltpu.make_async_remote_copy(
                src_ref=stage_a, dst_ref=rs_comm_a,
                send_sem=send_sem_a, recv_sem=recv_sem_a,
                device_id=(right,), device_id_type=pl.DeviceIdType.MESH,
            )
            rdma_b = pltpu.make_async_remote_copy(
                src_ref=stage_b, dst_ref=rs_comm_b,
                send_sem=send_sem_b, recv_sem=recv_sem_b,
                device_id=(left,), device_id_type=pl.DeviceIdType.MESH,
            )
            rdma_a.start()
            rdma_b.start()
            rdma_a.wait()
            rdma_b.wait()

            out_ref[pl.ds(recv_a * ch, ch), :n2] = (
                out_ref[pl.ds(recv_a * ch, ch), :n2]
                + rs_comm_a[...].astype(jnp.float32)
            )
            out_ref[pl.ds(recv_b * ch, ch), n2:] = (
                out_ref[pl.ds(recv_b * ch, ch), n2:]
                + rs_comm_b[...].astype(jnp.float32)
            )

            @pl.when(s < N_DEV - 2)
            def _():
                pl.semaphore_signal(
                    credit_a, inc=1, device_id=(left,),
                    device_id_type=pl.DeviceIdType.MESH,
                )
                pl.semaphore_signal(
                    credit_b, inc=1, device_id=(right,),
                    device_id_type=pl.DeviceIdType.MESH,
                )
            return _

        lax.fori_loop(0, N_DEV - 1, rs_hop, None)

        own_a = jnp.mod(my + 1, N_DEV)
        own_b = jnp.mod(my - 1, N_DEV)
        scale = sx_ref[0] * sw_ref[0]
        rows_a = pl.ds(own_a * ch, ch)
        rows_b = pl.ds(own_b * ch, ch)
        out_ref[rows_a, :n2] = jnp.maximum(out_ref[rows_a, :n2] * scale, 0.0)
        out_ref[rows_b, n2:] = jnp.maximum(out_ref[rows_b, n2:] * scale, 0.0)
        stage_a[...] = out_ref[rows_a, :n2].astype(jnp.bfloat16)
        stage_b[...] = out_ref[rows_b, n2:].astype(jnp.bfloat16)

        def ag_hop(s, _):
            recv_a = jnp.mod(my - s, N_DEV)
            recv_b = jnp.mod(my + s, N_DEV)
            slot = jnp.mod(s, 2)
            prev_slot = jnp.mod(s - 1, 2)

            @pl.when(s > 0)
            def _():
                pl.semaphore_wait(credit_a, 1)
                pl.semaphore_wait(credit_b, 1)

            def start_hop(src_a, src_b):
                rdma_a = pltpu.make_async_remote_copy(
                    src_ref=src_a, dst_ref=ag_comm_a.at[slot],
                    send_sem=send_sem_a, recv_sem=recv_sem_a,
                    device_id=(right,), device_id_type=pl.DeviceIdType.MESH,
                )
                rdma_b = pltpu.make_async_remote_copy(
                    src_ref=src_b, dst_ref=ag_comm_b.at[slot],
                    send_sem=send_sem_b, recv_sem=recv_sem_b,
                    device_id=(left,), device_id_type=pl.DeviceIdType.MESH,
                )
                rdma_a.start()
                rdma_b.start()
                rdma_a.wait()
                rdma_b.wait()

            @pl.when(s == 0)
            def _():
                start_hop(stage_a, stage_b)

            @pl.when(s > 0)
            def _():
                start_hop(ag_comm_a.at[prev_slot], ag_comm_b.at[prev_slot])

            out_ref[pl.ds(recv_a * ch, ch), :n2] = (
                ag_comm_a[slot].astype(jnp.float32))
            out_ref[pl.ds(recv_b * ch, ch), n2:] = (
                ag_comm_b[slot].astype(jnp.float32))

            @pl.when(s < N_DEV - 2)
            def _():
                pl.semaphore_signal(
                    credit_a, inc=1, device_id=(left,),
                    device_id_type=pl.DeviceIdType.MESH,
                )
                pl.semaphore_signal(
                    credit_b, inc=1, device_id=(right,),
                    device_id_type=pl.DeviceIdType.MESH,
                )
            return _

        lax.fori_loop(0, N_DEV - 1, ag_hop, None)

    return pl.pallas_call(
        body,
        out_shape=jax.ShapeDtypeStruct((m, n), jnp.float32),
        in_specs=[
            pl.BlockSpec(memory_space=pltpu.VMEM),
            pl.BlockSpec(memory_space=pltpu.VMEM),
            pl.BlockSpec(memory_space=pltpu.SMEM),
            pl.BlockSpec(memory_space=pltpu.SMEM),
        ],
        out_specs=pl.BlockSpec(memory_space=pltpu.VMEM),
        scratch_shapes=[
            pltpu.VMEM((ch, n2), jnp.bfloat16),
            pltpu.VMEM((ch, n2), jnp.bfloat16),
            pltpu.VMEM((ch, n2), jnp.bfloat16),
            pltpu.VMEM((ch, n2), jnp.bfloat16),
            pltpu.VMEM((2, ch, n2), jnp.bfloat16),
            pltpu.VMEM((2, ch, n2), jnp.bfloat16),
            pltpu.SemaphoreType.DMA,
            pltpu.SemaphoreType.DMA,
            pltpu.SemaphoreType.DMA,
            pltpu.SemaphoreType.DMA,
            pltpu.SemaphoreType.REGULAR,
            pltpu.SemaphoreType.REGULAR,
        ],
        compiler_params=pltpu.CompilerParams(
            collective_id=0,
            vmem_limit_bytes=60 * 1024 * 1024,
        ),
    )(x, w_mat, scale_x, scale_w)


# device time: 382954 ns/iter; 3.0668x vs baseline; 1.7368x over previous
import jax
import jax.numpy as jnp
import numpy as np
from jax import lax
from jax.experimental import pallas as pl
from jax.experimental.pallas import tpu as pltpu

N_DEV = 32

_CYCLE = np.array(
    [1, 2, 5, 6, 14, 13, 10, 9, 17, 18, 21, 22, 30, 29, 26, 25,
     24, 27, 28, 31, 23, 20, 19, 16, 8, 11, 12, 15, 7, 4, 3, 0],
    dtype=np.int32,
)
_CYCLE_INV = np.argsort(_CYCLE).astype(np.int32)


def kernel(x, w_mat, scale_x, scale_w):
    m, k = x.shape
    _, n = w_mat.shape
    ch = m // N_DEV
    n2 = n // 2

    def body(x_ref, w_ref, sx_ref, sw_ref, ids_ref, out_ref,
             stage_a, stage_b, rs_comm_a, rs_comm_b, ag_comm_a, ag_comm_b,
             send_sem_a, recv_sem_a, send_sem_b, recv_sem_b,
             credit_a, credit_b):
        my = ids_ref[0]
        left = ids_ref[1]
        right = ids_ref[2]

        wb = w_ref[...].astype(jnp.bfloat16)

        def gemm_chunk(c, _):
            out_ref[pl.ds(c * ch, ch), :] = jnp.dot(
                x_ref[pl.ds(c * ch, ch), :].astype(jnp.bfloat16),
                wb,
                preferred_element_type=jnp.float32,
            )
            return _

        lax.fori_loop(0, N_DEV, gemm_chunk, None)

        bar = pltpu.get_barrier_semaphore()
        for nbr in (left, right):
            pl.semaphore_signal(
                bar, inc=1, device_id=(nbr,),
                device_id_type=pl.DeviceIdType.MESH,
            )
        pl.semaphore_wait(bar, 2)

        def rs_hop(s, _):
            send_a = jnp.mod(my - s, N_DEV)
            recv_a = jnp.mod(my - s - 1, N_DEV)
            send_b = jnp.mod(my + s, N_DEV)
            recv_b = jnp.mod(my + s + 1, N_DEV)

            stage_a[...] = out_ref[pl.ds(send_a * ch, ch), :n2].astype(
                jnp.bfloat16)
            stage_b[...] = out_ref[pl.ds(send_b * ch, ch), n2:].astype(
                jnp.bfloat16)

            @pl.when(s > 0)
            def _():
                pl.semaphore_wait(credit_a, 1)
                pl.semaphore_wait(credit_b, 1)

            rdma_a = pltpu.make_async_remote_copy(
                src_ref=stage_a, dst_ref=rs_comm_a,
                send_sem=send_sem_a, recv_sem=recv_sem_a,
                device_id=(right,), device_id_type=pl.DeviceIdType.MESH,
            )
            rdma_b = pltpu.make_async_remote_copy(
                src_ref=stage_b, dst_ref=rs_comm_b,
                send_sem=send_sem_b, recv_sem=recv_sem_b,
                device_id=(left,), device_id_type=pl.DeviceIdType.MESH,
            )
            rdma_a.start()
            rdma_b.start()
            rdma_a.wait()
            rdma_b.wait()

            out_ref[pl.ds(recv_a * ch, ch), :n2] = (
                out_ref[pl.ds(recv_a * ch, ch), :n2]
                + rs_comm_a[...].astype(jnp.float32)
            )
            out_ref[pl.ds(recv_b * ch, ch), n2:] = (
                out_ref[pl.ds(recv_b * ch, ch), n2:]
                + rs_comm_b[...].astype(jnp.float32)
            )

            @pl.when(s < N_DEV - 2)
            def _():
                pl.semaphore_signal(
                    credit_a, inc=1, device_id=(left,),
                    device_id_type=pl.DeviceIdType.MESH,
                )
                pl.semaphore_signal(
                    credit_b, inc=1, device_id=(right,),
                    device_id_type=pl.DeviceIdType.MESH,
                )
            return _

        lax.fori_loop(0, N_DEV - 1, rs_hop, None)

        own_a = jnp.mod(my + 1, N_DEV)
        own_b = jnp.mod(my - 1, N_DEV)
        scale = sx_ref[0] * sw_ref[0]
        rows_a = pl.ds(own_a * ch, ch)
        rows_b = pl.ds(own_b * ch, ch)
        out_ref[rows_a, :n2] = jnp.maximum(out_ref[rows_a, :n2] * scale, 0.0)
        out_ref[rows_b, n2:] = jnp.maximum(out_ref[rows_b, n2:] * scale, 0.0)
        stage_a[...] = out_ref[rows_a, :n2].astype(jnp.bfloat16)
        stage_b[...] = out_ref[rows_b, n2:].astype(jnp.bfloat16)

        def ag_hop(s, _):
            recv_a = jnp.mod(my - s, N_DEV)
            recv_b = jnp.mod(my + s, N_DEV)
            slot = jnp.mod(s, 2)
            prev_slot = jnp.mod(s - 1, 2)

            @pl.when(s > 0)
            def _():
                pl.semaphore_wait(credit_a, 1)
                pl.semaphore_wait(credit_b, 1)

            def start_hop(src_a, src_b):
                rdma_a = pltpu.make_async_remote_copy(
                    src_ref=src_a, dst_ref=ag_comm_a.at[slot],
                    send_sem=send_sem_a, recv_sem=recv_sem_a,
                    device_id=(right,), device_id_type=pl.DeviceIdType.MESH,
                )
                rdma_b = pltpu.make_async_remote_copy(
                    src_ref=src_b, dst_ref=ag_comm_b.at[slot],
                    send_sem=send_sem_b, recv_sem=recv_sem_b,
                    device_id=(left,), device_id_type=pl.DeviceIdType.MESH,
                )
                rdma_a.start()
                rdma_b.start()
                rdma_a.wait()
                rdma_b.wait()

            @pl.when(s == 0)
            def _():
                start_hop(stage_a, stage_b)

            @pl.when(s > 0)
            def _():
                start_hop(ag_comm_a.at[prev_slot], ag_comm_b.at[prev_slot])

            out_ref[pl.ds(recv_a * ch, ch), :n2] = (
                ag_comm_a[slot].astype(jnp.float32))
            out_ref[pl.ds(recv_b * ch, ch), n2:] = (
                ag_comm_b[slot].astype(jnp.float32))

            @pl.when(s < N_DEV - 2)
            def _():
                pl.semaphore_signal(
                    credit_a, inc=1, device_id=(left,),
                    device_id_type=pl.DeviceIdType.MESH,
                )
                pl.semaphore_signal(
                    credit_b, inc=1, device_id=(right,),
                    device_id_type=pl.DeviceIdType.MESH,
                )
            return _

        lax.fori_loop(0, N_DEV - 1, ag_hop, None)

    mesh_idx = lax.axis_index("i")
    rho = jnp.take(jnp.asarray(_CYCLE_INV), mesh_idx)
    cyc = jnp.asarray(_CYCLE)
    left = jnp.take(cyc, jnp.mod(rho - 1, N_DEV))
    right = jnp.take(cyc, jnp.mod(rho + 1, N_DEV))
    ids = jnp.stack([rho, left, right]).astype(jnp.int32)

    return pl.pallas_call(
        body,
        out_shape=jax.ShapeDtypeStruct((m, n), jnp.float32),
        in_specs=[
            pl.BlockSpec(memory_space=pltpu.VMEM),
            pl.BlockSpec(memory_space=pltpu.VMEM),
            pl.BlockSpec(memory_space=pltpu.SMEM),
            pl.BlockSpec(memory_space=pltpu.SMEM),
            pl.BlockSpec(memory_space=pltpu.SMEM),
        ],
        out_specs=pl.BlockSpec(memory_space=pltpu.VMEM),
        scratch_shapes=[
            pltpu.VMEM((ch, n2), jnp.bfloat16),
            pltpu.VMEM((ch, n2), jnp.bfloat16),
            pltpu.VMEM((ch, n2), jnp.bfloat16),
            pltpu.VMEM((ch, n2), jnp.bfloat16),
            pltpu.VMEM((2, ch, n2), jnp.bfloat16),
            pltpu.VMEM((2, ch, n2), jnp.bfloat16),
            pltpu.SemaphoreType.DMA,
            pltpu.SemaphoreType.DMA,
            pltpu.SemaphoreType.DMA,
            pltpu.SemaphoreType.DMA,
            pltpu.SemaphoreType.REGULAR,
            pltpu.SemaphoreType.REGULAR,
        ],
        compiler_params=pltpu.CompilerParams(
            collective_id=0,
            vmem_limit_bytes=60 * 1024 * 1024,
        ),
    )(x, w_mat, scale_x, scale_w, ids)


# device time: 339797 ns/iter; 3.4563x vs baseline; 1.1270x over previous
import jax
import jax.numpy as jnp
import numpy as np
from jax import lax
from jax.experimental import pallas as pl
from jax.experimental.pallas import tpu as pltpu

N_DEV = 32

_CYCLE = np.array(
    [1, 2, 5, 6, 14, 13, 10, 9, 17, 18, 21, 22, 30, 29, 26, 25,
     24, 27, 28, 31, 23, 20, 19, 16, 8, 11, 12, 15, 7, 4, 3, 0],
    dtype=np.int32,
)
_CYCLE_INV = np.argsort(_CYCLE).astype(np.int32)


def kernel(x, w_mat, scale_x, scale_w):
    m, k = x.shape
    _, n = w_mat.shape
    ch = m // N_DEV
    n2 = n // 2

    def body(x_ref, w_ref, sx_ref, sw_ref, ids_ref, out_ref,
             stage_a, stage_b, rs_comm_a, rs_comm_b, ag_comm_a, ag_comm_b,
             send_sems_a, recv_sems_a, send_sems_b, recv_sems_b,
             credit_a, credit_b):
        rho = ids_ref[0]
        left = ids_ref[1]
        right = ids_ref[2]

        def desc_a(src, dst, par):
            return pltpu.make_async_remote_copy(
                src_ref=src, dst_ref=dst,
                send_sem=send_sems_a.at[par], recv_sem=recv_sems_a.at[par],
                device_id=(right,), device_id_type=pl.DeviceIdType.MESH,
            )

        def desc_b(src, dst, par):
            return pltpu.make_async_remote_copy(
                src_ref=src, dst_ref=dst,
                send_sem=send_sems_b.at[par], recv_sem=recv_sems_b.at[par],
                device_id=(left,), device_id_type=pl.DeviceIdType.MESH,
            )

        wb = w_ref[...].astype(jnp.bfloat16)

        def gemm_chunk(c, _):
            out_ref[pl.ds(c * ch, ch), :] = jnp.dot(
                x_ref[pl.ds(c * ch, ch), :].astype(jnp.bfloat16),
                wb,
                preferred_element_type=jnp.float32,
            )
            return _

        lax.fori_loop(0, N_DEV, gemm_chunk, None)

        bar = pltpu.get_barrier_semaphore()
        for nbr in (left, right):
            pl.semaphore_signal(
                bar, inc=1, device_id=(nbr,),
                device_id_type=pl.DeviceIdType.MESH,
            )
        pl.semaphore_wait(bar, 2)

        stage_a[0] = out_ref[pl.ds(rho * ch, ch), :n2].astype(jnp.bfloat16)
        stage_b[0] = out_ref[pl.ds(rho * ch, ch), n2:].astype(jnp.bfloat16)
        desc_a(stage_a.at[0], rs_comm_a.at[0], 0).start()
        desc_b(stage_b.at[0], rs_comm_b.at[0], 0).start()

        def rs_hop(s, _):
            par = jnp.mod(s, 2)
            nxt = jnp.mod(s + 1, 2)
            ra = jnp.mod(rho - s - 1, N_DEV)
            rb = jnp.mod(rho + s + 1, N_DEV)

            desc_a(stage_a.at[par], rs_comm_a.at[par], par).wait_recv()
            desc_b(stage_b.at[par], rs_comm_b.at[par], par).wait_recv()

            sum_a = (out_ref[pl.ds(ra * ch, ch), :n2]
                     + rs_comm_a[par].astype(jnp.float32))
            out_ref[pl.ds(ra * ch, ch), :n2] = sum_a
            sum_b = (out_ref[pl.ds(rb * ch, ch), n2:]
                     + rs_comm_b[par].astype(jnp.float32))
            out_ref[pl.ds(rb * ch, ch), n2:] = sum_b

            @pl.when(s < N_DEV - 2)
            def _():
                @pl.when(s >= 1)
                def _():
                    desc_a(stage_a.at[nxt], rs_comm_a.at[nxt],
                           nxt).wait_send()
                    desc_b(stage_b.at[nxt], rs_comm_b.at[nxt],
                           nxt).wait_send()
                    pl.semaphore_wait(credit_a, 1)
                    pl.semaphore_wait(credit_b, 1)
                stage_a[nxt] = sum_a.astype(jnp.bfloat16)
                stage_b[nxt] = sum_b.astype(jnp.bfloat16)
                desc_a(stage_a.at[nxt], rs_comm_a.at[nxt], nxt).start()
                desc_b(stage_b.at[nxt], rs_comm_b.at[nxt], nxt).start()

            @pl.when(s < N_DEV - 3)
            def _():
                pl.semaphore_signal(
                    credit_a, inc=1, device_id=(left,),
                    device_id_type=pl.DeviceIdType.MESH,
                )
                pl.semaphore_signal(
                    credit_b, inc=1, device_id=(right,),
                    device_id_type=pl.DeviceIdType.MESH,
                )
            return _

        lax.fori_loop(0, N_DEV - 1, rs_hop, None)

        desc_a(stage_a.at[1], rs_comm_a.at[1], 1).wait_send()
        desc_b(stage_b.at[1], rs_comm_b.at[1], 1).wait_send()
        desc_a(stage_a.at[0], rs_comm_a.at[0], 0).wait_send()
        desc_b(stage_b.at[0], rs_comm_b.at[0], 0).wait_send()

        own_a = jnp.mod(rho + 1, N_DEV)
        own_b = jnp.mod(rho - 1, N_DEV)
        scale = sx_ref[0] * sw_ref[0]
        rows_a = pl.ds(own_a * ch, ch)
        rows_b = pl.ds(own_b * ch, ch)
        out_ref[rows_a, :n2] = jnp.maximum(out_ref[rows_a, :n2] * scale, 0.0)
        out_ref[rows_b, n2:] = jnp.maximum(out_ref[rows_b, n2:] * scale, 0.0)
        stage_a[0] = out_ref[rows_a, :n2].astype(jnp.bfloat16)
        stage_b[0] = out_ref[rows_b, n2:].astype(jnp.bfloat16)

        desc_a(stage_a.at[0], ag_comm_a.at[0], 0).start()
        desc_b(stage_b.at[0], ag_comm_b.at[0], 0).start()

        def ag_hop(s, _):
            par = jnp.mod(s, 2)
            nxt = jnp.mod(s + 1, 2)
            ra = jnp.mod(rho - s, N_DEV)
            rb = jnp.mod(rho + s, N_DEV)

            desc_a(ag_comm_a.at[par], ag_comm_a.at[par], par).wait_recv()
            desc_b(ag_comm_b.at[par], ag_comm_b.at[par], par).wait_recv()

            @pl.when(s < N_DEV - 2)
            def _():
                @pl.when(s >= 1)
                def _():
                    pl.semaphore_wait(credit_a, 1)
                    pl.semaphore_wait(credit_b, 1)
                desc_a(ag_comm_a.at[par], ag_comm_a.at[nxt], nxt).start()
                desc_b(ag_comm_b.at[par], ag_comm_b.at[nxt], nxt).start()

            out_ref[pl.ds(ra * ch, ch), :n2] = (
                ag_comm_a[par].astype(jnp.float32))
            out_ref[pl.ds(rb * ch, ch), n2:] = (
                ag_comm_b[par].astype(jnp.float32))

            @pl.when(s < N_DEV - 2)
            def _():
                @pl.when(s == 0)
                def _():
                    desc_a(stage_a.at[0], ag_comm_a.at[0], 0).wait_send()
                    desc_b(stage_b.at[0], ag_comm_b.at[0], 0).wait_send()
                desc_a(ag_comm_a.at[par], ag_comm_a.at[nxt], nxt).wait_send()
                desc_b(ag_comm_b.at[par], ag_comm_b.at[nxt], nxt).wait_send()

            @pl.when(s < N_DEV - 3)
            def _():
                pl.semaphore_signal(
                    credit_a, inc=1, device_id=(left,),
                    device_id_type=pl.DeviceIdType.MESH,
                )
                pl.semaphore_signal(
                    credit_b, inc=1, device_id=(right,),
                    device_id_type=pl.DeviceIdType.MESH,
                )
            return _

        lax.fori_loop(0, N_DEV - 1, ag_hop, None)

    mesh_idx = lax.axis_index("i")
    rho = jnp.take(jnp.asarray(_CYCLE_INV), mesh_idx)
    cyc = jnp.asarray(_CYCLE)
    left = jnp.take(cyc, jnp.mod(rho - 1, N_DEV))
    right = jnp.take(cyc, jnp.mod(rho + 1, N_DEV))
    ids = jnp.stack([rho, left, right]).astype(jnp.int32)

    return pl.pallas_call(
        body,
        out_shape=jax.ShapeDtypeStruct((m, n), jnp.float32),
        in_specs=[
            pl.BlockSpec(memory_space=pltpu.VMEM),
            pl.BlockSpec(memory_space=pltpu.VMEM),
            pl.BlockSpec(memory_space=pltpu.SMEM),
            pl.BlockSpec(memory_space=pltpu.SMEM),
            pl.BlockSpec(memory_space=pltpu.SMEM),
        ],
        out_specs=pl.BlockSpec(memory_space=pltpu.VMEM),
        scratch_shapes=[
            pltpu.VMEM((2, ch, n2), jnp.bfloat16),
            pltpu.VMEM((2, ch, n2), jnp.bfloat16),
            pltpu.VMEM((2, ch, n2), jnp.bfloat16),
            pltpu.VMEM((2, ch, n2), jnp.bfloat16),
            pltpu.VMEM((2, ch, n2), jnp.bfloat16),
            pltpu.VMEM((2, ch, n2), jnp.bfloat16),
            pltpu.SemaphoreType.DMA((2,)),
            pltpu.SemaphoreType.DMA((2,)),
            pltpu.SemaphoreType.DMA((2,)),
            pltpu.SemaphoreType.DMA((2,)),
            pltpu.SemaphoreType.REGULAR,
            pltpu.SemaphoreType.REGULAR,
        ],
        compiler_params=pltpu.CompilerParams(
            collective_id=0,
            vmem_limit_bytes=60 * 1024 * 1024,
        ),
    )(x, w_mat, scale_x, scale_w, ids)
